# baseline (speedup 1.0000x reference)
import jax
import jax.numpy as jnp
from jax import lax
from jax.experimental import pallas as pl
from jax.experimental.pallas import tpu as pltpu

B = 256
EMB = 16
K = 1024
HID = 32
GEN = 65536
CB = 8192
NCB = GEN // CB


def _fused_kernel(z_ref, cb_ref, w1_ref, b1_ref, w2_ref, b2_ref, wk_hbm,
                  out_hbm, loss_ref, wk_buf, out_buf, in_sem, out_sem):
    def wk_copy(j, buf):
        return pltpu.make_async_copy(
            wk_hbm.at[:, pl.ds(j * CB, CB)], wk_buf.at[buf], in_sem.at[buf])

    def out_copy(j, buf):
        return pltpu.make_async_copy(
            out_buf.at[buf], out_hbm.at[:, pl.ds(j * CB, CB)],
            out_sem.at[buf])

    wk_copy(0, 0).start()
    wk_copy(1, 1).start()

    z = z_ref[...]
    cb = cb_ref[...]
    hT = lax.dot_general(cb, w1_ref[...], (((1,), (1,)), ((), ())),
                         preferred_element_type=jnp.float32) + b1_ref[...]
    hT = jnp.maximum(hT, 0.0)
    encT = lax.dot_general(hT, w2_ref[...], (((1,), (1,)), ((), ())),
                           preferred_element_type=jnp.float32) \
        + b2_ref[...]
    zz = z * z
    z2t = lax.dot_general(jnp.ones((1, EMB), jnp.float32), zz,
                          (((1,), (1,)), ((), ())),
                          preferred_element_type=jnp.float32)
    cb2 = jnp.sum(cb * cb, axis=1, keepdims=True)
    crosst = lax.dot_general(cb, z, (((1,), (1,)), ((), ())),
                             preferred_element_type=jnp.float32)
    dt = z2t - 2.0 * crosst + cb2
    dmint = jnp.min(dt, axis=0, keepdims=True)
    iit = lax.broadcasted_iota(jnp.int32, dt.shape, 0)
    idxt = jnp.min(jnp.where(dt == dmint, iit, jnp.int32(K)), axis=0,
                   keepdims=True)
    loss_ref[0, 0] = 1.25 * jnp.sum(dmint) / (B * EMB)
    onehott = (iit == idxt).astype(jnp.float32)
    enc = lax.dot_general(onehott, encT, (((0,), (0,)), ((), ())),
                          preferred_element_type=jnp.float32)

    NSUB = 4
    SUB = CB // NSUB
    OUTB = 3

    def sub_copy(c):
        return pltpu.make_async_copy(
            out_buf.at[0, :, pl.ds(c * SUB, SUB)],
            out_hbm.at[:, pl.ds(c * SUB, SUB)], out_sem.at[0])

    wk_copy(0, 0).wait()
    for c in range(NSUB):
        out_buf[0, :, pl.ds(c * SUB, SUB)] = lax.dot_general(
            enc, wk_buf[0, :, pl.ds(c * SUB, SUB)], (((1,), (0,)), ((), ())),
            preferred_element_type=jnp.float32)
        sub_copy(c).start()
    wk_copy(2, 0).start()

    for j in range(1, NCB):
        wbuf = j % 2
        slot = j % OUTB
        wk_copy(j, wbuf).wait()
        if j >= OUTB:
            prev = j - OUTB
            if prev == 0:
                for c in range(NSUB):
                    sub_copy(c).wait()
            else:
                out_copy(prev, prev % OUTB).wait()
        out_buf[slot] = lax.dot_general(
            enc, wk_buf[wbuf], (((1,), (0,)), ((), ())),
            preferred_element_type=jnp.float32)
        out_copy(j, slot).start()
        if j + 2 < NCB:
            wk_copy(j + 2, wbuf).start()
    for j in range(NCB - OUTB, NCB):
        out_copy(j, j % OUTB).wait()


@jax.jit
def kernel(z, codebook, W1, b1, W2, b2, Wk):
    gen, loss = pl.pallas_call(
        _fused_kernel,
        out_shape=(
            jax.ShapeDtypeStruct((B, GEN), jnp.float32),
            jax.ShapeDtypeStruct((1, 1), jnp.float32),
        ),
        in_specs=[
            pl.BlockSpec((B, EMB), lambda: (0, 0)),
            pl.BlockSpec((K, EMB), lambda: (0, 0)),
            pl.BlockSpec((HID, EMB), lambda: (0, 0)),
            pl.BlockSpec((1, HID), lambda: (0, 0)),
            pl.BlockSpec((EMB, HID), lambda: (0, 0)),
            pl.BlockSpec((1, EMB), lambda: (0, 0)),
            pl.BlockSpec(memory_space=pl.ANY),
        ],
        out_specs=(
            pl.BlockSpec(memory_space=pl.ANY),
            pl.BlockSpec(memory_space=pltpu.SMEM),
        ),
        scratch_shapes=[
            pltpu.VMEM((2, EMB, CB), jnp.float32),
            pltpu.VMEM((3, B, CB), jnp.float32),
            pltpu.SemaphoreType.DMA((2,)),
            pltpu.SemaphoreType.DMA((3,)),
        ],
    )(z, codebook, W1, b1.reshape(1, HID), W2, b2.reshape(1, EMB), Wk)
    return gen, loss[0, 0]


# speedup vs baseline: 1.0039x; 1.0039x over previous
import jax
import jax.numpy as jnp
from jax import lax
from jax.experimental import pallas as pl
from jax.experimental.pallas import tpu as pltpu

B = 256
EMB = 16
K = 1024
HID = 32
GEN = 65536
CB = 8192
NCB = GEN // CB


def _fused_kernel(z_ref, cb_ref, w1_ref, b1_ref, w2_ref, b2_ref, wk_hbm,
                  out_hbm, loss_ref, wk_buf, out_buf, in_sem, out_sem):
    def wk_copy(j, buf):
        return pltpu.make_async_copy(
            wk_hbm.at[:, pl.ds(j * CB, CB)], wk_buf.at[buf], in_sem.at[buf])

    def out_copy(j, buf):
        return pltpu.make_async_copy(
            out_buf.at[buf], out_hbm.at[:, pl.ds(j * CB, CB)],
            out_sem.at[buf])

    wk_copy(0, 0).start()
    wk_copy(1, 1).start()

    z = z_ref[...]
    cb = cb_ref[...]
    hT = lax.dot_general(cb, w1_ref[...], (((1,), (1,)), ((), ())),
                         preferred_element_type=jnp.float32) + b1_ref[...]
    hT = jnp.maximum(hT, 0.0)
    encT = lax.dot_general(hT, w2_ref[...], (((1,), (1,)), ((), ())),
                           preferred_element_type=jnp.float32) \
        + b2_ref[...]
    z2 = jnp.sum(z * z, axis=1, keepdims=True)
    cb2 = jnp.sum(cb * cb, axis=1, keepdims=True)
    cross = lax.dot_general(z, cb, (((1,), (1,)), ((), ())),
                            preferred_element_type=jnp.float32)
    d = z2 - 2.0 * cross + cb2.T
    dmin = jnp.min(d, axis=1, keepdims=True)
    ii = lax.broadcasted_iota(jnp.int32, d.shape, 1)
    idx = jnp.min(jnp.where(d == dmin, ii, jnp.int32(K)), axis=1,
                  keepdims=True)
    loss_ref[0, 0] = 1.25 * jnp.sum(dmin) / (B * EMB)
    onehot = (ii == idx).astype(jnp.float32)
    enc = lax.dot_general(onehot, encT, (((1,), (0,)), ((), ())),
                          preferred_element_type=jnp.float32)

    NSUB = 4
    SUB = CB // NSUB
    OUTB = 3

    def sub_copy(c):
        return pltpu.make_async_copy(
            out_buf.at[0, :, pl.ds(c * SUB, SUB)],
            out_hbm.at[:, pl.ds(c * SUB, SUB)], out_sem.at[0])

    wk_copy(0, 0).wait()
    for c in range(NSUB):
        out_buf[0, :, pl.ds(c * SUB, SUB)] = lax.dot_general(
            enc, wk_buf[0, :, pl.ds(c * SUB, SUB)], (((1,), (0,)), ((), ())),
            preferred_element_type=jnp.float32)
        sub_copy(c).start()
    wk_copy(2, 0).start()

    for j in range(1, NCB):
        wbuf = j % 2
        slot = j % OUTB
        wk_copy(j, wbuf).wait()
        if j >= OUTB:
            prev = j - OUTB
            if prev == 0:
                for c in range(NSUB):
                    sub_copy(c).wait()
            else:
                out_copy(prev, prev % OUTB).wait()
        out_buf[slot] = lax.dot_general(
            enc, wk_buf[wbuf], (((1,), (0,)), ((), ())),
            preferred_element_type=jnp.float32)
        out_copy(j, slot).start()
        if j + 2 < NCB:
            wk_copy(j + 2, wbuf).start()
    for j in range(NCB - OUTB, NCB):
        out_copy(j, j % OUTB).wait()


@jax.jit
def kernel(z, codebook, W1, b1, W2, b2, Wk):
    gen, loss = pl.pallas_call(
        _fused_kernel,
        out_shape=(
            jax.ShapeDtypeStruct((B, GEN), jnp.float32),
            jax.ShapeDtypeStruct((1, 1), jnp.float32),
        ),
        in_specs=[
            pl.BlockSpec((B, EMB), lambda: (0, 0)),
            pl.BlockSpec((K, EMB), lambda: (0, 0)),
            pl.BlockSpec((HID, EMB), lambda: (0, 0)),
            pl.BlockSpec((1, HID), lambda: (0, 0)),
            pl.BlockSpec((EMB, HID), lambda: (0, 0)),
            pl.BlockSpec((1, EMB), lambda: (0, 0)),
            pl.BlockSpec(memory_space=pl.ANY),
        ],
        out_specs=(
            pl.BlockSpec(memory_space=pl.ANY),
            pl.BlockSpec(memory_space=pltpu.SMEM),
        ),
        scratch_shapes=[
            pltpu.VMEM((2, EMB, CB), jnp.float32),
            pltpu.VMEM((3, B, CB), jnp.float32),
            pltpu.SemaphoreType.DMA((2,)),
            pltpu.SemaphoreType.DMA((3,)),
        ],
    )(z, codebook, W1, b1.reshape(1, HID), W2, b2.reshape(1, EMB), Wk)
    return gen, loss[0, 0]
